# baseline (speedup 1.0000x reference)
import functools

import jax
import jax.numpy as jnp
import numpy as np
from jax.experimental import pallas as pl
from jax.experimental.pallas import tpu as pltpu

B = 1024
F = 256
E = 256
S = 128
DEPTH = 6
I = 2 ** DEPTH - 1
IP = 2 ** DEPTH
L = 2 ** DEPTH

EB = 16
GRID = E // EB

_HI = jax.lax.Precision.HIGHEST


def _bitrev(v, nbits):
    r = 0
    for _ in range(nbits):
        r = (r << 1) | (v & 1)
        v >>= 1
    return r


def _node_perm():
    perm = np.zeros(I, dtype=np.int32)
    for d in range(DEPTH):
        base = 2 ** d - 1
        for q in range(2 ** d):
            perm[base + q] = base + _bitrev(q, d)
    return perm


_NODE_PERM = _node_perm()
_LEAF_PERM = np.array([_bitrev(q, DEPTH) for q in range(L)], dtype=np.int32)
_NODE_PERM_PAD = np.tile(
    np.concatenate([_NODE_PERM, np.array([I], np.int32)])[:, None], (1, 128))


def _entmax15_rows(z):
    z = z * 0.5
    z = z - jnp.max(z, axis=-1, keepdims=True)
    lo = jnp.full(z.shape[:-1] + (1,), -1.0, dtype=z.dtype)
    hi = jnp.zeros_like(lo)
    for _ in range(20):
        mid = 0.5 * (lo + hi)
        f = jnp.sum(jnp.square(jnp.maximum(z - mid, 0.0)), axis=-1, keepdims=True)
        gt = f > 1.0
        lo = jnp.where(gt, mid, lo)
        hi = jnp.where(gt, hi, mid)
    tau0 = 0.5 * (lo + hi)
    mask = (z > tau0).astype(z.dtype)
    k = jnp.sum(mask, axis=-1, keepdims=True)
    mean = jnp.sum(z * mask, axis=-1, keepdims=True) / k
    mean_sq = jnp.sum(z * z * mask, axis=-1, keepdims=True) / k
    delta = (1.0 - k * (mean_sq - mean * mean)) / k
    tau = mean - jnp.sqrt(jnp.maximum(delta, 0.0))
    return jnp.square(jnp.maximum(z - tau, 0.0))


def _main_kernel(logits_ref, sv_ref, feats_ref, lcew_ref, xb_ref, perm_ref,
                 ye_ref, g_ref, w_scratch):
    sel = _entmax15_rows(logits_ref[...])
    s2 = jnp.sum(sel * sv_ref[...], axis=-1, keepdims=True)
    sel_b = sel.astype(jnp.bfloat16).astype(jnp.float32)

    piota = jax.lax.broadcasted_iota(jnp.int32, (IP, IP), 1)
    pmat = (perm_ref[...][:, :IP] == piota).astype(jnp.float32)

    fiota = jax.lax.broadcasted_iota(jnp.int32, (F, S), 0)
    zrow = jnp.zeros((1, S), jnp.float32)
    s2cols = []
    for j in range(EB):
        feats_row = feats_ref[j:j + 1, :]
        oht = (fiota == feats_row).astype(jnp.float32)
        sel_j = jnp.concatenate(
            [sel_b[j * I:(j + 1) * I, :], zrow], axis=0)
        sel_pj = jax.lax.dot_general(
            pmat, sel_j, (((1,), (0,)), ((), ())),
            precision=_HI, preferred_element_type=jnp.float32)
        w_scratch[j * IP:(j + 1) * IP, :] = jax.lax.dot_general(
            sel_pj, oht, (((1,), (1,)), ((), ())),
            precision=_HI, preferred_element_type=jnp.float32)
        s2cols.append(jnp.concatenate(
            [s2[j * I:(j + 1) * I, :], zrow[:, 0:1]], axis=0))
    s2p = jax.lax.dot_general(
        pmat, jnp.concatenate(s2cols, axis=1), (((1,), (0,)), ((), ())),
        precision=_HI, preferred_element_type=jnp.float32)

    s1 = jnp.dot(w_scratch[...], xb_ref[...],
                 precision=_HI, preferred_element_type=jnp.float32)

    for j in range(EB):
        base = j * IP
        bits = ((s1[base:base + IP, :] - s2p[:, j:j + 1]) > 0.0
                ).astype(jnp.float32)
        p = None
        for d in range(DEPTH):
            lv = bits[2 ** d - 1: 2 ** (d + 1) - 1, :]
            if p is None:
                p = jnp.concatenate([1.0 - lv, lv], axis=0)
            else:
                p = jnp.concatenate([p * (1.0 - lv), p * lv], axis=0)
        pdup = jnp.concatenate([p, p], axis=0)
        yg = jax.lax.dot_general(
            lcew_ref[2 * j:2 * j + 2, :], pdup, (((1,), (0,)), ((), ())),
            precision=_HI, preferred_element_type=jnp.float32)
        ye_ref[j:j + 1, :] = yg[0:1, :]
        g_ref[j:j + 1, :] = yg[1:2, :]


def _ensemble_kernel(g_ref, ye_ref, out_ref):
    g = g_ref[...]
    z = g * 0.5
    z = z - jnp.max(z, axis=0, keepdims=True)
    lo = jnp.full((1, z.shape[1]), -1.0, dtype=z.dtype)
    hi = jnp.zeros_like(lo)
    for _ in range(20):
        mid = 0.5 * (lo + hi)
        f = jnp.sum(jnp.square(jnp.maximum(z - mid, 0.0)), axis=0, keepdims=True)
        gt = f > 1.0
        lo = jnp.where(gt, mid, lo)
        hi = jnp.where(gt, hi, mid)
    tau0 = 0.5 * (lo + hi)
    mask = (z > tau0).astype(z.dtype)
    k = jnp.sum(mask, axis=0, keepdims=True)
    mean = jnp.sum(z * mask, axis=0, keepdims=True) / k
    mean_sq = jnp.sum(z * z * mask, axis=0, keepdims=True) / k
    delta = (1.0 - k * (mean_sq - mean * mean)) / k
    tau = mean - jnp.sqrt(jnp.maximum(delta, 0.0))
    w = jnp.square(jnp.maximum(z - tau, 0.0))
    out_ref[...] = jnp.sum(w * ye_ref[...], axis=0, keepdims=True)


@jax.jit
def _run(x, split_values, split_index_logits, estimator_weights, leaf_classes,
         features_by_estimator):
    lperm = jnp.asarray(_LEAF_PERM)
    permarr = jnp.asarray(_NODE_PERM_PAD)
    lg = split_index_logits.reshape(E * I, S)
    sv = split_values.reshape(E * I, S)
    lcr = leaf_classes[:, lperm]
    ewr = estimator_weights[:, lperm]
    zpad = jnp.zeros((E, L), jnp.float32)
    lcew = jnp.stack(
        [jnp.concatenate([lcr, zpad], axis=1),
         jnp.concatenate([zpad, ewr], axis=1)], axis=1).reshape(2 * E, 2 * L)
    xb = x.astype(jnp.bfloat16).astype(jnp.float32).T

    ye, g = pl.pallas_call(
        _main_kernel,
        grid=(GRID,),
        in_specs=[
            pl.BlockSpec((EB * I, S), lambda i: (i, 0)),
            pl.BlockSpec((EB * I, S), lambda i: (i, 0)),
            pl.BlockSpec((EB, S), lambda i: (i, 0)),
            pl.BlockSpec((2 * EB, 2 * L), lambda i: (i, 0)),
            pl.BlockSpec((F, B), lambda i: (0, 0)),
            pl.BlockSpec((IP, 128), lambda i: (0, 0)),
        ],
        out_specs=[
            pl.BlockSpec((EB, B), lambda i: (i, 0)),
            pl.BlockSpec((EB, B), lambda i: (i, 0)),
        ],
        out_shape=[
            jax.ShapeDtypeStruct((E, B), jnp.float32),
            jax.ShapeDtypeStruct((E, B), jnp.float32),
        ],
        scratch_shapes=[pltpu.VMEM((EB * IP, F), jnp.float32)],
    )(lg, sv, features_by_estimator, lcew, xb, permarr)

    out = pl.pallas_call(
        _ensemble_kernel,
        in_specs=[
            pl.BlockSpec((E, B), lambda: (0, 0)),
            pl.BlockSpec((E, B), lambda: (0, 0)),
        ],
        out_specs=pl.BlockSpec((1, B), lambda: (0, 0)),
        out_shape=jax.ShapeDtypeStruct((1, B), jnp.float32),
    )(g, ye)
    return out.reshape(B)


def kernel(x, split_values, split_index_logits, estimator_weights,
           leaf_classes, features_by_estimator, internal_node_index,
           path_identifier):
    del internal_node_index, path_identifier
    return _run(x, split_values, split_index_logits, estimator_weights,
                leaf_classes, features_by_estimator)


# speedup vs baseline: 1.0416x; 1.0416x over previous
import functools

import jax
import jax.numpy as jnp
import numpy as np
from jax.experimental import pallas as pl
from jax.experimental.pallas import tpu as pltpu

B = 1024
F = 256
E = 256
S = 128
DEPTH = 6
I = 2 ** DEPTH - 1
IP = 2 ** DEPTH
L = 2 ** DEPTH

EB = 16
GRID = E // EB

_HI = jax.lax.Precision.HIGHEST


def _bitrev(v, nbits):
    r = 0
    for _ in range(nbits):
        r = (r << 1) | (v & 1)
        v >>= 1
    return r


def _node_perm():
    perm = np.zeros(I, dtype=np.int32)
    for d in range(DEPTH):
        base = 2 ** d - 1
        for q in range(2 ** d):
            perm[base + q] = base + _bitrev(q, d)
    return perm


_NODE_PERM = _node_perm()
_LEAF_PERM = np.array([_bitrev(q, DEPTH) for q in range(L)], dtype=np.int32)


def _entmax15_rows(z):
    z = z * 0.5
    z = z - jnp.max(z, axis=-1, keepdims=True)
    lo = jnp.full(z.shape[:-1] + (1,), -1.0, dtype=z.dtype)
    hi = jnp.zeros_like(lo)
    for _ in range(20):
        mid = 0.5 * (lo + hi)
        f = jnp.sum(jnp.square(jnp.maximum(z - mid, 0.0)), axis=-1, keepdims=True)
        gt = f > 1.0
        lo = jnp.where(gt, mid, lo)
        hi = jnp.where(gt, hi, mid)
    tau0 = 0.5 * (lo + hi)
    mask = (z > tau0).astype(z.dtype)
    k = jnp.sum(mask, axis=-1, keepdims=True)
    mean = jnp.sum(z * mask, axis=-1, keepdims=True) / k
    mean_sq = jnp.sum(z * z * mask, axis=-1, keepdims=True) / k
    delta = (1.0 - k * (mean_sq - mean * mean)) / k
    tau = mean - jnp.sqrt(jnp.maximum(delta, 0.0))
    return jnp.square(jnp.maximum(z - tau, 0.0))


def _main_kernel(logits_ref, sv_ref, feats_ref, lcew_ref, xb_ref,
                 ye_ref, g_ref, w_scratch):
    sel = _entmax15_rows(logits_ref[...])
    s2 = jnp.sum(sel * sv_ref[...], axis=-1, keepdims=True)
    sel_b = sel.astype(jnp.bfloat16).astype(jnp.float32)

    fiota = jax.lax.broadcasted_iota(jnp.int32, (F, S), 0)
    for j in range(EB):
        feats_row = feats_ref[j:j + 1, :]
        oht = (fiota == feats_row).astype(jnp.float32)
        sel_j = sel_b[j * IP:(j + 1) * IP, :]
        w_scratch[j * IP:(j + 1) * IP, :] = jax.lax.dot_general(
            sel_j, oht, (((1,), (1,)), ((), ())),
            precision=_HI, preferred_element_type=jnp.float32)

    s1 = jnp.dot(w_scratch[...], xb_ref[...],
                 precision=_HI, preferred_element_type=jnp.float32)
    bits = ((s1 - s2) > 0.0).astype(jnp.float32)

    for j in range(EB):
        base = j * IP
        p = None
        for d in range(DEPTH):
            lv = bits[base + 2 ** d - 1: base + 2 ** (d + 1) - 1, :]
            if p is None:
                p = jnp.concatenate([1.0 - lv, lv], axis=0)
            else:
                p = jnp.concatenate([p * (1.0 - lv), p * lv], axis=0)
        pdup = jnp.concatenate([p, p], axis=0)
        yg = jax.lax.dot_general(
            lcew_ref[2 * j:2 * j + 2, :], pdup, (((1,), (0,)), ((), ())),
            precision=_HI, preferred_element_type=jnp.float32)
        ye_ref[j:j + 1, :] = yg[0:1, :]
        g_ref[j:j + 1, :] = yg[1:2, :]


def _ensemble_kernel(g_ref, ye_ref, out_ref):
    g = g_ref[...]
    z = g * 0.5
    z = z - jnp.max(z, axis=0, keepdims=True)
    lo = jnp.full((1, z.shape[1]), -1.0, dtype=z.dtype)
    hi = jnp.zeros_like(lo)
    for _ in range(20):
        mid = 0.5 * (lo + hi)
        f = jnp.sum(jnp.square(jnp.maximum(z - mid, 0.0)), axis=0, keepdims=True)
        gt = f > 1.0
        lo = jnp.where(gt, mid, lo)
        hi = jnp.where(gt, hi, mid)
    tau0 = 0.5 * (lo + hi)
    mask = (z > tau0).astype(z.dtype)
    k = jnp.sum(mask, axis=0, keepdims=True)
    mean = jnp.sum(z * mask, axis=0, keepdims=True) / k
    mean_sq = jnp.sum(z * z * mask, axis=0, keepdims=True) / k
    delta = (1.0 - k * (mean_sq - mean * mean)) / k
    tau = mean - jnp.sqrt(jnp.maximum(delta, 0.0))
    w = jnp.square(jnp.maximum(z - tau, 0.0))
    out_ref[...] = jnp.sum(w * ye_ref[...], axis=0, keepdims=True)


@jax.jit
def _run(x, split_values, split_index_logits, estimator_weights, leaf_classes,
         features_by_estimator):
    perm = jnp.asarray(_NODE_PERM)
    lperm = jnp.asarray(_LEAF_PERM)
    lg = split_index_logits[:, perm, :]
    sv = split_values[:, perm, :]
    lg = jnp.pad(lg, ((0, 0), (0, IP - I), (0, 0))).reshape(E * IP, S)
    sv = jnp.pad(sv, ((0, 0), (0, IP - I), (0, 0))).reshape(E * IP, S)
    lcr = leaf_classes[:, lperm]
    ewr = estimator_weights[:, lperm]
    zpad = jnp.zeros((E, L), jnp.float32)
    lcew = jnp.stack(
        [jnp.concatenate([lcr, zpad], axis=1),
         jnp.concatenate([zpad, ewr], axis=1)], axis=1).reshape(2 * E, 2 * L)
    xb = x.astype(jnp.bfloat16).astype(jnp.float32).T

    ye, g = pl.pallas_call(
        _main_kernel,
        grid=(GRID,),
        in_specs=[
            pl.BlockSpec((EB * IP, S), lambda i: (i, 0)),
            pl.BlockSpec((EB * IP, S), lambda i: (i, 0)),
            pl.BlockSpec((EB, S), lambda i: (i, 0)),
            pl.BlockSpec((2 * EB, 2 * L), lambda i: (i, 0)),
            pl.BlockSpec((F, B), lambda i: (0, 0)),
        ],
        out_specs=[
            pl.BlockSpec((EB, B), lambda i: (i, 0)),
            pl.BlockSpec((EB, B), lambda i: (i, 0)),
        ],
        out_shape=[
            jax.ShapeDtypeStruct((E, B), jnp.float32),
            jax.ShapeDtypeStruct((E, B), jnp.float32),
        ],
        scratch_shapes=[pltpu.VMEM((EB * IP, F), jnp.float32)],
    )(lg, sv, features_by_estimator, lcew, xb)

    out = pl.pallas_call(
        _ensemble_kernel,
        in_specs=[
            pl.BlockSpec((E, B), lambda: (0, 0)),
            pl.BlockSpec((E, B), lambda: (0, 0)),
        ],
        out_specs=pl.BlockSpec((1, B), lambda: (0, 0)),
        out_shape=jax.ShapeDtypeStruct((1, B), jnp.float32),
    )(g, ye)
    return out.reshape(B)


def kernel(x, split_values, split_index_logits, estimator_weights,
           leaf_classes, features_by_estimator, internal_node_index,
           path_identifier):
    del internal_node_index, path_identifier
    return _run(x, split_values, split_index_logits, estimator_weights,
                leaf_classes, features_by_estimator)
